# Initial kernel scaffold; baseline (speedup 1.0000x reference)
import jax
import jax.numpy as jnp
from jax.experimental import pallas as pl


def kernel(preds, anchors):
    raise NotImplementedError("write your pallas kernel here")


# speedup vs baseline: 9.4089x; 9.4089x over previous
import functools

import jax
import jax.numpy as jnp
from jax.experimental import pallas as pl
from jax.experimental.pallas import tpu as pltpu

SCORE_THRESHOLD = 0.3
IOU_THRESHOLD = 0.5
MAX_OUTPUT_SIZE = 100
LANES = 128


def _nms_kernel(preds_ref, anchors_ref, out_ref,
                sc_ref, y1_ref, x1_ref, y2_ref, x2_ref, lb_ref,
                *, n, rows):
    chunk = LANES
    n_full = n // chunk
    tail = n - n_full * chunk

    def decode_chunk(start, size, row):
        p = preds_ref[0, pl.ds(start, size), :]
        pt = p.T
        a = anchors_ref[:, pl.ds(row, 1), :][:, 0, :size]
        cls = pt[4:, :]
        ccs = jnp.max(cls, axis=0)
        nc = cls.shape[0]
        srow = jax.lax.broadcasted_iota(jnp.int32, cls.shape, 0)
        lbl = jnp.min(jnp.where(cls == ccs[None, :], srow, nc), axis=0)
        ahw0 = a[2] - a[0]
        ahw1 = a[3] - a[1]
        ac0 = (a[0] + a[2]) * 0.5
        ac1 = (a[1] + a[3]) * 0.5
        c0 = pt[0] * ahw0 + ac0
        c1 = pt[1] * ahw1 + ac1
        h0 = jnp.exp(pt[2]) * ahw0
        h1 = jnp.exp(pt[3]) * ahw1
        t0 = c0 - 0.5 * h0
        t1 = c1 - 0.5 * h1
        b0 = t0 + h0
        b1 = t1 + h1
        score = jnp.where(ccs > SCORE_THRESHOLD, ccs, -1.0)
        sz = pl.ds(0, size)
        sc_ref[row, sz] = score
        y1_ref[row, sz] = t0
        x1_ref[row, sz] = t1
        y2_ref[row, sz] = b0
        x2_ref[row, sz] = b1
        lb_ref[row, sz] = lbl.astype(jnp.float32)

    def body(i, _):
        decode_chunk(i * chunk, chunk, i)
        return 0

    jax.lax.fori_loop(0, n_full, body, 0)
    if tail:
        sc_ref[n_full, :] = jnp.full((LANES,), -2.0, jnp.float32)
        decode_chunk(n_full * chunk, tail, n_full)

    flat = (jax.lax.broadcasted_iota(jnp.int32, (rows, LANES), 0) * LANES
            + jax.lax.broadcasted_iota(jnp.int32, (rows, LANES), 1))
    big = jnp.int32(rows * LANES)

    def nms_iter(i, _):
        s = sc_ref[...]
        m = jnp.max(s)
        idx = jnp.min(jnp.where(s == m, flat, big))
        valid = m > 0.0
        onehot = flat == idx
        by1 = jnp.sum(jnp.where(onehot, y1_ref[...], 0.0))
        bx1 = jnp.sum(jnp.where(onehot, x1_ref[...], 0.0))
        by2 = jnp.sum(jnp.where(onehot, y2_ref[...], 0.0))
        bx2 = jnp.sum(jnp.where(onehot, x2_ref[...], 0.0))
        blb = jnp.sum(jnp.where(onehot, lb_ref[...], 0.0))
        it0 = jnp.maximum(by1, y1_ref[...])
        it1 = jnp.maximum(bx1, x1_ref[...])
        ib0 = jnp.minimum(by2, y2_ref[...])
        ib1 = jnp.minimum(bx2, x2_ref[...])
        ih = jnp.maximum(ib0 - it0, 0.0)
        iw = jnp.maximum(ib1 - it1, 0.0)
        inter = ih * iw
        a1 = (by2 - by1) * (bx2 - bx1)
        a2 = (y2_ref[...] - y1_ref[...]) * (x2_ref[...] - x1_ref[...])
        iou = inter / (a1 + a2 - inter + 1e-9)
        suppress = ((iou > IOU_THRESHOLD) & valid) | onehot
        sc_ref[...] = jnp.where(suppress, -1.0, s)
        vf = valid.astype(jnp.float32)
        rowv = jnp.stack([by1, bx1, by2, bx2, blb, m]) * vf
        out_ref[0, pl.ds(i, 1), :] = rowv.reshape(1, 6)
        return 0

    jax.lax.fori_loop(0, MAX_OUTPUT_SIZE, nms_iter, 0)


@jax.jit
def kernel(preds, anchors):
    batch, n, _ = preds.shape
    rows = (n + LANES - 1) // LANES
    npad = rows * LANES
    anc = jnp.pad(anchors, ((0, npad - n), (0, 0)))
    anc = anc.T.reshape(4, rows, LANES)
    out = pl.pallas_call(
        functools.partial(_nms_kernel, n=n, rows=rows),
        grid=(batch,),
        in_specs=[
            pl.BlockSpec((1, n, preds.shape[2]), lambda b: (b, 0, 0)),
            pl.BlockSpec((4, rows, LANES), lambda b: (0, 0, 0)),
        ],
        out_specs=pl.BlockSpec((1, MAX_OUTPUT_SIZE, 6), lambda b: (b, 0, 0)),
        out_shape=jax.ShapeDtypeStruct((batch, MAX_OUTPUT_SIZE, 6),
                                       jnp.float32),
        scratch_shapes=[pltpu.VMEM((rows, LANES), jnp.float32)
                        for _ in range(6)],
    )(preds, anc)
    return out


# speedup vs baseline: 13.4504x; 1.4295x over previous
import functools

import jax
import jax.numpy as jnp
from jax import lax
from jax.experimental import pallas as pl
from jax.experimental.pallas import tpu as pltpu
from jax.experimental.pallas import tpu_sc as plsc

SCORE_THRESHOLD = 0.3
IOU_THRESHOLD = 0.5
MAX_OUTPUT_SIZE = 100
LANES = 128
N = 49104
ROWS = 384
NPAD = ROWS * LANES
NSUB = 8
SLICE = NPAD // NSUB
VPB = SLICE // 16
CAP = 1024
CAPP = 2 * CAP + 64
TARGET = 192
BISECT_ROUNDS = 30


def _iota16():
    return lax.broadcasted_iota(jnp.int32, (16,), 0)


def _splat_f(x):
    return jnp.full((16,), x, jnp.float32)


def _splat_i(x):
    return jnp.full((16,), x, jnp.int32)


def _decode_kernel(preds_ref, anchors_ref,
                   sc_o, y1_o, x1_o, y2_o, x2_o, lb_o):
    n_full = N // LANES
    tail = N - n_full * LANES

    def decode_chunk(start, size, row):
        p = preds_ref[0, pl.ds(start, size), :]
        pt = p.T
        a = anchors_ref[:, pl.ds(row, 1), :][:, 0, :size]
        cls = pt[4:, :]
        ccs = jnp.max(cls, axis=0)
        nc = cls.shape[0]
        srow = lax.broadcasted_iota(jnp.int32, cls.shape, 0)
        lbl = jnp.min(jnp.where(cls == ccs[None, :], srow, nc), axis=0)
        ahw0 = a[2] - a[0]
        ahw1 = a[3] - a[1]
        ac0 = (a[0] + a[2]) * 0.5
        ac1 = (a[1] + a[3]) * 0.5
        c0 = pt[0] * ahw0 + ac0
        c1 = pt[1] * ahw1 + ac1
        h0 = jnp.exp(pt[2]) * ahw0
        h1 = jnp.exp(pt[3]) * ahw1
        t0 = c0 - 0.5 * h0
        t1 = c1 - 0.5 * h1
        score = jnp.where(ccs > SCORE_THRESHOLD, ccs, -1.0)
        sz = pl.ds(0, size)
        sc_o[0, row, sz] = score
        y1_o[0, row, sz] = t0
        x1_o[0, row, sz] = t1
        y2_o[0, row, sz] = b0 = t0 + h0
        x2_o[0, row, sz] = b1 = t1 + h1
        del b0, b1
        lb_o[0, row, sz] = lbl.astype(jnp.float32)

    def body(i, _):
        decode_chunk(i * LANES, LANES, i)
        return 0

    lax.fori_loop(0, n_full, body, 0)
    if tail:
        sc_o[0, n_full, :] = jnp.full((LANES,), -2.0, jnp.float32)
        decode_chunk(n_full * LANES, tail, n_full)


def _tc_decode(preds, anc):
    batch = preds.shape[0]
    outs = pl.pallas_call(
        _decode_kernel,
        grid=(batch,),
        in_specs=[
            pl.BlockSpec((1, N, preds.shape[2]), lambda b: (b, 0, 0)),
            pl.BlockSpec((4, ROWS, LANES), lambda b: (0, 0, 0)),
        ],
        out_specs=[pl.BlockSpec((1, ROWS, LANES), lambda b: (b, 0, 0))
                   for _ in range(6)],
        out_shape=[jax.ShapeDtypeStruct((batch, ROWS, LANES), jnp.float32)
                   for _ in range(6)],
    )(preds, anc)
    return outs


def _sc_kernel(sc_h, y1_h, x1_h, y2_h, x2_h, lb_h,
               out_h, flags_h,
               sc_v, y1_v, x1_v, y2_v, x2_v, lb_v,
               pool_v, packed_v, p1_v, p2_v, p3_v, p4_v, p5_v,
               outbuf_v, cnt_v, cnt8_v, flg_v,
               partial_sh, pool_sh, sem):
    c = lax.axis_index("c")
    s = lax.axis_index("s")
    batch = c * 2 + s // NSUB
    b2 = s // NSUB
    slot = s % NSUB
    base = pl.multiple_of(batch * NPAD + slot * SLICE, 8)
    iota = _iota16()

    cp1 = pltpu.async_copy(y1_h.at[pl.ds(base, SLICE)], y1_v, sem)
    cp2 = pltpu.async_copy(x1_h.at[pl.ds(base, SLICE)], x1_v, sem)
    cp3 = pltpu.async_copy(y2_h.at[pl.ds(base, SLICE)], y2_v, sem)
    cp4 = pltpu.async_copy(x2_h.at[pl.ds(base, SLICE)], x2_v, sem)
    cp5 = pltpu.async_copy(lb_h.at[pl.ds(base, SLICE)], lb_v, sem)
    pltpu.sync_copy(sc_h.at[pl.ds(base, SLICE)], sc_v)

    def local_count(t):
        def cbody(i, acc):
            v = sc_v[pl.ds(i * 16, 16)]
            return acc + jnp.where(v > t, 1.0, 0.0)
        acc = lax.fori_loop(0, VPB, cbody, jnp.zeros((16,), jnp.float32))
        return jnp.sum(acc)

    def global_count(t):
        cnt = local_count(t)
        cnt_v[...] = _splat_f(cnt)
        pltpu.sync_copy(cnt_v, partial_sh.at[pl.ds(pl.multiple_of(s * 16, 8), 16)])
        plsc.subcore_barrier()
        pltpu.sync_copy(partial_sh.at[pl.ds(pl.multiple_of(b2 * NSUB * 16, 8), NSUB * 16)], cnt8_v)
        total = jnp.float32(0)
        for j in range(NSUB):
            total = total + jnp.max(cnt8_v[pl.ds(j * 16, 16)])
        plsc.subcore_barrier()
        return total

    positives = global_count(jnp.float32(SCORE_THRESHOLD))

    def bisect(_, carry):
        lo, hi = carry
        mid = (lo + hi) * 0.5
        tot = global_count(mid)
        ge = tot >= TARGET
        lo = jnp.where(ge, mid, lo)
        hi = jnp.where(ge, hi, mid)
        return lo, hi

    lo, _ = lax.fori_loop(0, BISECT_ROUNDS, bisect,
                          (jnp.float32(SCORE_THRESHOLD), jnp.float32(1.0)))
    c_total = global_count(lo)
    overflow = (c_total > CAP).astype(jnp.int32)


    cp1.wait()
    cp2.wait()
    cp3.wait()
    cp4.wait()
    cp5.wait()

    def compact(i, off):
        sl = pl.ds(i * 16, 16)
        sv = sc_v[sl]
        mask = sv > lo
        ones = jnp.where(mask, 1.0, 0.0)
        cum = plsc.cumsum(ones)
        pos = (cum + (off - 1.0)).astype(jnp.int32)
        wmask = mask & (pos < CAP)
        vals = (sv, y1_v[sl], x1_v[sl], y2_v[sl], x2_v[sl], lb_v[sl])
        for r in range(6):
            plsc.store_scatter(pool_v, [_splat_i(r * CAP) + pos], vals[r],
                               mask=wmask)
        return off + jnp.max(cum)

    cntf = lax.fori_loop(0, VPB, compact, jnp.float32(0))
    cnt = jnp.minimum(cntf.astype(jnp.int32), CAP)
    padded = (cnt + 7) & ~7
    padidx = cnt + iota
    plsc.store_scatter(pool_v, [padidx], _splat_f(-3.0),
                       mask=(padidx < padded) & (padidx < CAP))
    for r in range(6):
        pltpu.sync_copy(pool_v.at[pl.ds(r * CAP, CAP)],
                        pool_sh.at[pl.ds(pl.multiple_of(
                            ((b2 * NSUB + slot) * 6 + r) * CAP, 8), CAP)])
    cnt_v[...] = _splat_f(padded.astype(jnp.float32))
    pltpu.sync_copy(cnt_v, partial_sh.at[pl.ds(pl.multiple_of(s * 16, 8), 16)])
    plsc.subcore_barrier()

    @pl.when((slot == 0) & (overflow == 1))
    def _flag_only():
        flg_v[...] = _splat_i(jnp.int32(1))
        pltpu.sync_copy(flg_v.at[pl.ds(0, 8)],
                        flags_h.at[pl.ds(pl.multiple_of(batch * 8, 8), 8)])

    @pl.when((slot == 0) & (overflow == 0))
    def _nms():
        pltpu.sync_copy(partial_sh.at[pl.ds(pl.multiple_of(b2 * NSUB * 16, 8), NSUB * 16)], cnt8_v)
        def initp(i, _):
            packed_v[pl.ds(i * 16, 16)] = _splat_f(-3.0)
            return 0
        lax.fori_loop(0, CAPP // 16, initp, 0)
        offf = jnp.float32(0)
        for j in range(NSUB):
            pj = jnp.max(cnt8_v[pl.ds(j * 16, 16)])
            packs = (packed_v, p1_v, p2_v, p3_v, p4_v, p5_v)
            off = pl.multiple_of(offf.astype(jnp.int32), 8)
            for r in range(6):
                pltpu.sync_copy(
                    pool_sh.at[pl.ds(pl.multiple_of(
                        ((b2 * NSUB + j) * 6 + r) * CAP, 8), CAP)],
                    packs[r].at[pl.ds(off, CAP)])
            offf = offf + pj
        total_i = offf.astype(jnp.int32)
        tidx = total_i + iota
        plsc.store_scatter(packed_v, [tidx], _splat_f(-3.0),
                           mask=tidx < CAPP)
        nv = (total_i + 15) // 16

        def nms_iter(i, carry):
            nsel, _of = carry

            iota_f = iota.astype(jnp.float32)

            def amax(j, acc):
                run, rid = acc
                sl = pl.ds(j * 16, 16)
                v = packed_v[sl]
                pos = jnp.full((16,), (j * 16).astype(jnp.float32),
                               jnp.float32) + iota_f
                gt = v > run
                return jnp.where(gt, v, run), jnp.where(gt, pos, rid)

            run, rid = lax.fori_loop(
                0, nv, amax, (_splat_f(-1e30), _splat_f(0.0)))
            m = jnp.max(run)
            p = jnp.min(jnp.where(run == m, rid,
                                  jnp.float32(CAPP))).astype(jnp.int32)
            p = jnp.minimum(p, CAPP - 1)
            valid = m > 0.0
            pv = _splat_i(p)
            by1 = plsc.load_gather(p1_v, [pv])
            bx1 = plsc.load_gather(p2_v, [pv])
            by2 = plsc.load_gather(p3_v, [pv])
            bx2 = plsc.load_gather(p4_v, [pv])
            blb = plsc.load_gather(p5_v, [pv])
            a1 = (by2 - by1) * (bx2 - bx1)

            def supp(j, _):
                sl = pl.ds(j * 16, 16)
                sv = packed_v[sl]
                t0 = jnp.maximum(by1, y1p := p1_v[sl])
                t1 = jnp.maximum(bx1, x1p := p2_v[sl])
                b0 = jnp.minimum(by2, y2p := p3_v[sl])
                b1 = jnp.minimum(bx2, x2p := p4_v[sl])
                ih = jnp.maximum(b0 - t0, 0.0)
                iw = jnp.maximum(b1 - t1, 0.0)
                inter = ih * iw
                a2 = (y2p - y1p) * (x2p - x1p)
                iou = inter / (a1 + a2 - inter + 1e-9)
                pos = j * 16 + iota
                kill = ((iou > IOU_THRESHOLD) & valid) | (pos == p)
                packed_v[sl] = jnp.where(kill, -1.0, sv)
                return 0

            lax.fori_loop(0, nv, supp, 0)
            vf = jnp.where(valid, 1.0, 0.0)
            row = (jnp.where(iota == 0, by1, 0.0)
                   + jnp.where(iota == 1, bx1, 0.0)
                   + jnp.where(iota == 2, by2, 0.0)
                   + jnp.where(iota == 3, bx2, 0.0)
                   + jnp.where(iota == 4, blb, 0.0)
                   + jnp.where(iota == 5, _splat_f(m), 0.0)) * vf
            plsc.store_scatter(outbuf_v, [_splat_i(i * 6) + iota], row,
                               mask=iota < 6)
            return nsel + jnp.where(valid, 1, 0).astype(jnp.int32), _of

        nsel, _ = lax.fori_loop(0, MAX_OUTPUT_SIZE, nms_iter,
                                (jnp.int32(0), jnp.int32(0)))
        flag = ((nsel < MAX_OUTPUT_SIZE)
                & (positives > c_total)).astype(jnp.int32)
        pltpu.sync_copy(outbuf_v, out_h.at[pl.ds(pl.multiple_of(batch * 600, 8), 600)])
        flg_v[...] = _splat_i(flag)
        pltpu.sync_copy(flg_v.at[pl.ds(0, 8)], flags_h.at[pl.ds(pl.multiple_of(batch * 8, 8), 8)])


def _sc_select_nms(flats):
    mesh = plsc.VectorSubcoreMesh(core_axis_name="c", subcore_axis_name="s")
    kfn = functools.partial(
        pl.kernel, mesh=mesh,
        compiler_params=pltpu.CompilerParams(needs_layout_passes=False),
        out_type=[jax.ShapeDtypeStruct((4 * MAX_OUTPUT_SIZE * 6,),
                                       jnp.float32),
                  jax.ShapeDtypeStruct((4 * 8,), jnp.int32)],
        scratch_types=[pltpu.VMEM((SLICE,), jnp.float32) for _ in range(6)]
        + [pltpu.VMEM((6 * CAP,), jnp.float32)]
        + [pltpu.VMEM((CAPP,), jnp.float32) for _ in range(6)]
        + [pltpu.VMEM((MAX_OUTPUT_SIZE * 6,), jnp.float32),
           pltpu.VMEM((16,), jnp.float32),
           pltpu.VMEM((NSUB * 16,), jnp.float32),
           pltpu.VMEM((16,), jnp.int32),
           pltpu.VMEM_SHARED((16 * 16,), jnp.float32),
           pltpu.VMEM_SHARED((2 * NSUB * 6 * CAP,), jnp.float32),
           pltpu.SemaphoreType.DMA],
    )(_sc_kernel)
    return kfn(*flats)


def _full_nms_kernel(preds_ref, anchors_ref, out_ref,
                     sc_ref, y1_ref, x1_ref, y2_ref, x2_ref, lb_ref):
    n_full = N // LANES
    tail = N - n_full * LANES

    def decode_chunk(start, size, row):
        p = preds_ref[0, pl.ds(start, size), :]
        pt = p.T
        a = anchors_ref[:, pl.ds(row, 1), :][:, 0, :size]
        cls = pt[4:, :]
        ccs = jnp.max(cls, axis=0)
        nc = cls.shape[0]
        srow = lax.broadcasted_iota(jnp.int32, cls.shape, 0)
        lbl = jnp.min(jnp.where(cls == ccs[None, :], srow, nc), axis=0)
        ahw0 = a[2] - a[0]
        ahw1 = a[3] - a[1]
        ac0 = (a[0] + a[2]) * 0.5
        ac1 = (a[1] + a[3]) * 0.5
        c0 = pt[0] * ahw0 + ac0
        c1 = pt[1] * ahw1 + ac1
        h0 = jnp.exp(pt[2]) * ahw0
        h1 = jnp.exp(pt[3]) * ahw1
        t0 = c0 - 0.5 * h0
        t1 = c1 - 0.5 * h1
        score = jnp.where(ccs > SCORE_THRESHOLD, ccs, -1.0)
        sz = pl.ds(0, size)
        sc_ref[row, sz] = score
        y1_ref[row, sz] = t0
        x1_ref[row, sz] = t1
        y2_ref[row, sz] = t0 + h0
        x2_ref[row, sz] = t1 + h1
        lb_ref[row, sz] = lbl.astype(jnp.float32)

    def body(i, _):
        decode_chunk(i * LANES, LANES, i)
        return 0

    lax.fori_loop(0, n_full, body, 0)
    if tail:
        sc_ref[n_full, :] = jnp.full((LANES,), -2.0, jnp.float32)
        decode_chunk(n_full * LANES, tail, n_full)

    flat = (lax.broadcasted_iota(jnp.int32, (ROWS, LANES), 0) * LANES
            + lax.broadcasted_iota(jnp.int32, (ROWS, LANES), 1))
    big = jnp.int32(ROWS * LANES)

    def nms_iter(i, _):
        sv = sc_ref[...]
        m = jnp.max(sv)
        idx = jnp.min(jnp.where(sv == m, flat, big))
        valid = m > 0.0
        onehot = flat == idx
        by1 = jnp.sum(jnp.where(onehot, y1_ref[...], 0.0))
        bx1 = jnp.sum(jnp.where(onehot, x1_ref[...], 0.0))
        by2 = jnp.sum(jnp.where(onehot, y2_ref[...], 0.0))
        bx2 = jnp.sum(jnp.where(onehot, x2_ref[...], 0.0))
        blb = jnp.sum(jnp.where(onehot, lb_ref[...], 0.0))
        it0 = jnp.maximum(by1, y1_ref[...])
        it1 = jnp.maximum(bx1, x1_ref[...])
        ib0 = jnp.minimum(by2, y2_ref[...])
        ib1 = jnp.minimum(bx2, x2_ref[...])
        ih = jnp.maximum(ib0 - it0, 0.0)
        iw = jnp.maximum(ib1 - it1, 0.0)
        inter = ih * iw
        a1 = (by2 - by1) * (bx2 - bx1)
        a2 = (y2_ref[...] - y1_ref[...]) * (x2_ref[...] - x1_ref[...])
        iou = inter / (a1 + a2 - inter + 1e-9)
        suppress = ((iou > IOU_THRESHOLD) & valid) | onehot
        sc_ref[...] = jnp.where(suppress, -1.0, sv)
        vf = valid.astype(jnp.float32)
        rowv = jnp.stack([by1, bx1, by2, bx2, blb, m]) * vf
        out_ref[0, pl.ds(i, 1), :] = rowv.reshape(1, 6)
        return 0

    lax.fori_loop(0, MAX_OUTPUT_SIZE, nms_iter, 0)


def _full_nms(preds, anc):
    batch = preds.shape[0]
    return pl.pallas_call(
        _full_nms_kernel,
        grid=(batch,),
        in_specs=[
            pl.BlockSpec((1, N, preds.shape[2]), lambda b: (b, 0, 0)),
            pl.BlockSpec((4, ROWS, LANES), lambda b: (0, 0, 0)),
        ],
        out_specs=pl.BlockSpec((1, MAX_OUTPUT_SIZE, 6), lambda b: (b, 0, 0)),
        out_shape=jax.ShapeDtypeStruct((batch, MAX_OUTPUT_SIZE, 6),
                                       jnp.float32),
        scratch_shapes=[pltpu.VMEM((ROWS, LANES), jnp.float32)
                        for _ in range(6)],
    )(preds, anc)


@jax.jit
def kernel(preds, anchors):
    anc = jnp.pad(anchors, ((0, NPAD - N), (0, 0))).T.reshape(4, ROWS, LANES)
    decoded = _tc_decode(preds, anc)
    flats = [a.reshape(-1) for a in decoded]
    sc_out, flags = _sc_select_nms(flats)
    sc_out = sc_out.reshape(4, MAX_OUTPUT_SIZE, 6)
    ok = jnp.all(flags.reshape(4, 8)[:, 0] == 0)
    return lax.cond(ok,
                    lambda ops: ops[0],
                    lambda ops: _full_nms(ops[1], ops[2]),
                    (sc_out, preds, anc))


# speedup vs baseline: 18.8061x; 1.3982x over previous
import functools

import jax
import jax.numpy as jnp
from jax import lax
from jax.experimental import pallas as pl
from jax.experimental.pallas import tpu as pltpu
from jax.experimental.pallas import tpu_sc as plsc

SCORE_THRESHOLD = 0.3
IOU_THRESHOLD = 0.5
MAX_OUTPUT_SIZE = 100
LANES = 128
N = 49104
ROWS = 384
NPAD = ROWS * LANES
NSUB = 8
SLICE = NPAD // NSUB
VPB = SLICE // 16
CAP = 1024
CAPP = 2 * CAP + 64
TARGET = 192
BISECT_ROUNDS = 14


def _iota16():
    return lax.broadcasted_iota(jnp.int32, (16,), 0)


def _splat_f(x):
    return jnp.full((16,), x, jnp.float32)


def _splat_i(x):
    return jnp.full((16,), x, jnp.int32)


def _decode_kernel(preds_ref, anchors_ref,
                   sc_o, y1_o, x1_o, y2_o, x2_o, lb_o):
    n_full = N // LANES
    tail = N - n_full * LANES

    def decode_chunk(start, size, row):
        p = preds_ref[0, pl.ds(start, size), :]
        pt = p.T
        a = anchors_ref[:, pl.ds(row, 1), :][:, 0, :size]
        cls = pt[4:, :]
        ccs = jnp.max(cls, axis=0)
        nc = cls.shape[0]
        srow = lax.broadcasted_iota(jnp.int32, cls.shape, 0)
        lbl = jnp.min(jnp.where(cls == ccs[None, :], srow, nc), axis=0)
        ahw0 = a[2] - a[0]
        ahw1 = a[3] - a[1]
        ac0 = (a[0] + a[2]) * 0.5
        ac1 = (a[1] + a[3]) * 0.5
        c0 = pt[0] * ahw0 + ac0
        c1 = pt[1] * ahw1 + ac1
        h0 = jnp.exp(pt[2]) * ahw0
        h1 = jnp.exp(pt[3]) * ahw1
        t0 = c0 - 0.5 * h0
        t1 = c1 - 0.5 * h1
        score = jnp.where(ccs > SCORE_THRESHOLD, ccs, -1.0)
        sz = pl.ds(0, size)
        sc_o[0, row, sz] = score
        y1_o[0, row, sz] = t0
        x1_o[0, row, sz] = t1
        y2_o[0, row, sz] = b0 = t0 + h0
        x2_o[0, row, sz] = b1 = t1 + h1
        del b0, b1
        lb_o[0, row, sz] = lbl.astype(jnp.float32)

    def body(i, _):
        for u in range(4):
            decode_chunk((i * 4 + u) * LANES, LANES, i * 4 + u)
        return 0

    lax.fori_loop(0, n_full // 4, body, 0)
    for r in range((n_full // 4) * 4, n_full):
        decode_chunk(r * LANES, LANES, r)
    if tail:
        sc_o[0, n_full, :] = jnp.full((LANES,), -2.0, jnp.float32)
        decode_chunk(n_full * LANES, tail, n_full)


def _tc_decode(preds, anc):
    batch = preds.shape[0]
    outs = pl.pallas_call(
        _decode_kernel,
        grid=(batch,),
        in_specs=[
            pl.BlockSpec((1, N, preds.shape[2]), lambda b: (b, 0, 0)),
            pl.BlockSpec((4, ROWS, LANES), lambda b: (0, 0, 0)),
        ],
        out_specs=[pl.BlockSpec((1, ROWS, LANES), lambda b: (b, 0, 0))
                   for _ in range(6)],
        out_shape=[jax.ShapeDtypeStruct((batch, ROWS, LANES), jnp.float32)
                   for _ in range(6)],
    )(preds, anc)
    return outs


def _sc_kernel(sc_h, y1_h, x1_h, y2_h, x2_h, lb_h,
               out_h, flags_h,
               sc_v, y1_v, x1_v, y2_v, x2_v, lb_v,
               pool_v, packed_v, p1_v, p2_v, p3_v, p4_v, p5_v,
               outbuf_v, cnt_v, cnt8_v, flg_v,
               partial_sh, pool_sh, sem):
    c = lax.axis_index("c")
    s = lax.axis_index("s")
    batch = c * 2 + s // NSUB
    b2 = s // NSUB
    slot = s % NSUB
    base = pl.multiple_of(batch * NPAD + slot * SLICE, 8)
    iota = _iota16()

    cp1 = pltpu.async_copy(y1_h.at[pl.ds(base, SLICE)], y1_v, sem)
    cp2 = pltpu.async_copy(x1_h.at[pl.ds(base, SLICE)], x1_v, sem)
    cp3 = pltpu.async_copy(y2_h.at[pl.ds(base, SLICE)], y2_v, sem)
    cp4 = pltpu.async_copy(x2_h.at[pl.ds(base, SLICE)], x2_v, sem)
    cp5 = pltpu.async_copy(lb_h.at[pl.ds(base, SLICE)], lb_v, sem)
    pltpu.sync_copy(sc_h.at[pl.ds(base, SLICE)], sc_v)

    def local_count(t):
        def cbody(i, acc):
            v = sc_v[pl.ds(i * 16, 16)]
            return acc + jnp.where(v > t, 1.0, 0.0)
        acc = lax.fori_loop(0, VPB, cbody, jnp.zeros((16,), jnp.float32))
        return jnp.sum(acc)

    def publish(vec, r):
        cnt_v[...] = vec
        slot16 = pl.multiple_of((r * 16 + s) * 16, 8)
        pltpu.sync_copy(cnt_v, partial_sh.at[pl.ds(slot16, 16)])
        plsc.subcore_barrier()
        base16 = pl.multiple_of((r * 16 + b2 * NSUB) * 16, 8)
        pltpu.sync_copy(partial_sh.at[pl.ds(base16, NSUB * 16)], cnt8_v)

    def global_count(t, r):
        publish(_splat_f(local_count(t)), r)
        total = jnp.float32(0)
        for j in range(NSUB):
            total = total + jnp.max(cnt8_v[pl.ds(j * 16, 16)])
        return total

    positives = global_count(jnp.float32(SCORE_THRESHOLD), 0)

    def bisect(r, carry):
        lo, hi = carry
        mid = (lo + hi) * 0.5
        tot = global_count(mid, 1 + r)
        ge = tot >= TARGET
        lo = jnp.where(ge, mid, lo)
        hi = jnp.where(ge, hi, mid)
        return lo, hi

    lo, _ = lax.fori_loop(0, BISECT_ROUNDS, bisect,
                          (jnp.float32(SCORE_THRESHOLD), jnp.float32(1.0)))


    cp1.wait()
    cp2.wait()
    cp3.wait()
    cp4.wait()
    cp5.wait()

    def compact(i, off):
        sl = pl.ds(i * 16, 16)
        sv = sc_v[sl]
        mask = sv > lo
        ones = jnp.where(mask, 1.0, 0.0)
        cum = plsc.cumsum(ones)
        pos = (cum + (off - 1.0)).astype(jnp.int32)
        wmask = mask & (pos < CAP)
        vals = (sv, y1_v[sl], x1_v[sl], y2_v[sl], x2_v[sl], lb_v[sl])
        for r in range(6):
            plsc.store_scatter(pool_v, [_splat_i(r * CAP) + pos], vals[r],
                               mask=wmask)
        return off + jnp.max(cum)

    cntf = lax.fori_loop(0, VPB, compact, jnp.float32(0))
    cnt = jnp.minimum(cntf.astype(jnp.int32), CAP)
    padded = (cnt + 7) & ~7
    padidx = cnt + iota
    plsc.store_scatter(pool_v, [padidx], _splat_f(-3.0),
                       mask=(padidx < padded) & (padidx < CAP))
    for r in range(6):
        pltpu.sync_copy(pool_v.at[pl.ds(r * CAP, CAP)],
                        pool_sh.at[pl.ds(pl.multiple_of(
                            ((b2 * NSUB + slot) * 6 + r) * CAP, 8), CAP)])
    publish(jnp.where(iota < 8, cntf, padded.astype(jnp.float32)),
            1 + BISECT_ROUNDS)

    c_total = jnp.float32(0)
    for j in range(NSUB):
        row = cnt8_v[pl.ds(j * 16, 16)]
        c_total = c_total + jnp.max(jnp.where(iota < 8, row, -1.0))
    overflow = (c_total > CAP).astype(jnp.int32)

    @pl.when((slot == 0) & (overflow == 1))
    def _flag_only():
        flg_v[...] = _splat_i(jnp.int32(1))
        pltpu.sync_copy(flg_v.at[pl.ds(0, 8)],
                        flags_h.at[pl.ds(pl.multiple_of(batch * 8, 8), 8)])

    @pl.when((slot == 0) & (overflow == 0))
    def _nms():
        def initp(i, _):
            packed_v[pl.ds(i * 16, 16)] = _splat_f(-3.0)
            return 0
        lax.fori_loop(0, CAPP // 16, initp, 0)
        offf = jnp.float32(0)
        for j in range(NSUB):
            pj = jnp.max(jnp.where(iota >= 8,
                                   cnt8_v[pl.ds(j * 16, 16)], -1.0))
            packs = (packed_v, p1_v, p2_v, p3_v, p4_v, p5_v)
            off = pl.multiple_of(offf.astype(jnp.int32), 8)
            for r in range(6):
                pltpu.sync_copy(
                    pool_sh.at[pl.ds(pl.multiple_of(
                        ((b2 * NSUB + j) * 6 + r) * CAP, 8), CAP)],
                    packs[r].at[pl.ds(off, CAP)])
            offf = offf + pj
        total_i = offf.astype(jnp.int32)
        tidx = total_i + iota
        plsc.store_scatter(packed_v, [tidx], _splat_f(-3.0),
                           mask=tidx < CAPP)
        nv = (total_i + 15) // 16

        def nms_iter(i, carry):
            nsel, _of = carry

            iota_f = iota.astype(jnp.float32)

            def amax(j, acc):
                run, rid = acc
                sl = pl.ds(j * 16, 16)
                v = packed_v[sl]
                pos = jnp.full((16,), (j * 16).astype(jnp.float32),
                               jnp.float32) + iota_f
                gt = v > run
                return jnp.where(gt, v, run), jnp.where(gt, pos, rid)

            run, rid = lax.fori_loop(
                0, nv, amax, (_splat_f(-1e30), _splat_f(0.0)))
            m = jnp.max(run)
            p = jnp.min(jnp.where(run == m, rid,
                                  jnp.float32(CAPP))).astype(jnp.int32)
            p = jnp.minimum(p, CAPP - 1)
            valid = m > 0.0
            pv = _splat_i(p)
            by1 = plsc.load_gather(p1_v, [pv])
            bx1 = plsc.load_gather(p2_v, [pv])
            by2 = plsc.load_gather(p3_v, [pv])
            bx2 = plsc.load_gather(p4_v, [pv])
            blb = plsc.load_gather(p5_v, [pv])
            a1 = (by2 - by1) * (bx2 - bx1)

            def supp(j, _):
                sl = pl.ds(j * 16, 16)
                sv = packed_v[sl]
                t0 = jnp.maximum(by1, y1p := p1_v[sl])
                t1 = jnp.maximum(bx1, x1p := p2_v[sl])
                b0 = jnp.minimum(by2, y2p := p3_v[sl])
                b1 = jnp.minimum(bx2, x2p := p4_v[sl])
                ih = jnp.maximum(b0 - t0, 0.0)
                iw = jnp.maximum(b1 - t1, 0.0)
                inter = ih * iw
                a2 = (y2p - y1p) * (x2p - x1p)
                iou = inter / (a1 + a2 - inter + 1e-9)
                pos = j * 16 + iota
                kill = ((iou > IOU_THRESHOLD) & valid) | (pos == p)
                packed_v[sl] = jnp.where(kill, -1.0, sv)
                return 0

            lax.fori_loop(0, nv, supp, 0)
            vf = jnp.where(valid, 1.0, 0.0)
            row = (jnp.where(iota == 0, by1, 0.0)
                   + jnp.where(iota == 1, bx1, 0.0)
                   + jnp.where(iota == 2, by2, 0.0)
                   + jnp.where(iota == 3, bx2, 0.0)
                   + jnp.where(iota == 4, blb, 0.0)
                   + jnp.where(iota == 5, _splat_f(m), 0.0)) * vf
            plsc.store_scatter(outbuf_v, [_splat_i(i * 6) + iota], row,
                               mask=iota < 6)
            return nsel + jnp.where(valid, 1, 0).astype(jnp.int32), _of

        nsel, _ = lax.fori_loop(0, MAX_OUTPUT_SIZE, nms_iter,
                                (jnp.int32(0), jnp.int32(0)))
        flag = ((nsel < MAX_OUTPUT_SIZE)
                & (positives > c_total)).astype(jnp.int32)
        pltpu.sync_copy(outbuf_v, out_h.at[pl.ds(pl.multiple_of(batch * 600, 8), 600)])
        flg_v[...] = _splat_i(flag)
        pltpu.sync_copy(flg_v.at[pl.ds(0, 8)], flags_h.at[pl.ds(pl.multiple_of(batch * 8, 8), 8)])


def _sc_select_nms(flats):
    mesh = plsc.VectorSubcoreMesh(core_axis_name="c", subcore_axis_name="s")
    kfn = functools.partial(
        pl.kernel, mesh=mesh,
        compiler_params=pltpu.CompilerParams(needs_layout_passes=False),
        out_type=[jax.ShapeDtypeStruct((4 * MAX_OUTPUT_SIZE * 6,),
                                       jnp.float32),
                  jax.ShapeDtypeStruct((4 * 8,), jnp.int32)],
        scratch_types=[pltpu.VMEM((SLICE,), jnp.float32) for _ in range(6)]
        + [pltpu.VMEM((6 * CAP,), jnp.float32)]
        + [pltpu.VMEM((CAPP,), jnp.float32) for _ in range(6)]
        + [pltpu.VMEM((MAX_OUTPUT_SIZE * 6,), jnp.float32),
           pltpu.VMEM((16,), jnp.float32),
           pltpu.VMEM((NSUB * 16,), jnp.float32),
           pltpu.VMEM((16,), jnp.int32),
           pltpu.VMEM_SHARED(((BISECT_ROUNDS + 2) * 16 * 16,), jnp.float32),
           pltpu.VMEM_SHARED((2 * NSUB * 6 * CAP,), jnp.float32),
           pltpu.SemaphoreType.DMA],
    )(_sc_kernel)
    return kfn(*flats)


def _full_nms_kernel(preds_ref, anchors_ref, out_ref,
                     sc_ref, y1_ref, x1_ref, y2_ref, x2_ref, lb_ref):
    n_full = N // LANES
    tail = N - n_full * LANES

    def decode_chunk(start, size, row):
        p = preds_ref[0, pl.ds(start, size), :]
        pt = p.T
        a = anchors_ref[:, pl.ds(row, 1), :][:, 0, :size]
        cls = pt[4:, :]
        ccs = jnp.max(cls, axis=0)
        nc = cls.shape[0]
        srow = lax.broadcasted_iota(jnp.int32, cls.shape, 0)
        lbl = jnp.min(jnp.where(cls == ccs[None, :], srow, nc), axis=0)
        ahw0 = a[2] - a[0]
        ahw1 = a[3] - a[1]
        ac0 = (a[0] + a[2]) * 0.5
        ac1 = (a[1] + a[3]) * 0.5
        c0 = pt[0] * ahw0 + ac0
        c1 = pt[1] * ahw1 + ac1
        h0 = jnp.exp(pt[2]) * ahw0
        h1 = jnp.exp(pt[3]) * ahw1
        t0 = c0 - 0.5 * h0
        t1 = c1 - 0.5 * h1
        score = jnp.where(ccs > SCORE_THRESHOLD, ccs, -1.0)
        sz = pl.ds(0, size)
        sc_ref[row, sz] = score
        y1_ref[row, sz] = t0
        x1_ref[row, sz] = t1
        y2_ref[row, sz] = t0 + h0
        x2_ref[row, sz] = t1 + h1
        lb_ref[row, sz] = lbl.astype(jnp.float32)

    def body(i, _):
        decode_chunk(i * LANES, LANES, i)
        return 0

    lax.fori_loop(0, n_full, body, 0)
    if tail:
        sc_ref[n_full, :] = jnp.full((LANES,), -2.0, jnp.float32)
        decode_chunk(n_full * LANES, tail, n_full)

    flat = (lax.broadcasted_iota(jnp.int32, (ROWS, LANES), 0) * LANES
            + lax.broadcasted_iota(jnp.int32, (ROWS, LANES), 1))
    big = jnp.int32(ROWS * LANES)

    def nms_iter(i, _):
        sv = sc_ref[...]
        m = jnp.max(sv)
        idx = jnp.min(jnp.where(sv == m, flat, big))
        valid = m > 0.0
        onehot = flat == idx
        by1 = jnp.sum(jnp.where(onehot, y1_ref[...], 0.0))
        bx1 = jnp.sum(jnp.where(onehot, x1_ref[...], 0.0))
        by2 = jnp.sum(jnp.where(onehot, y2_ref[...], 0.0))
        bx2 = jnp.sum(jnp.where(onehot, x2_ref[...], 0.0))
        blb = jnp.sum(jnp.where(onehot, lb_ref[...], 0.0))
        it0 = jnp.maximum(by1, y1_ref[...])
        it1 = jnp.maximum(bx1, x1_ref[...])
        ib0 = jnp.minimum(by2, y2_ref[...])
        ib1 = jnp.minimum(bx2, x2_ref[...])
        ih = jnp.maximum(ib0 - it0, 0.0)
        iw = jnp.maximum(ib1 - it1, 0.0)
        inter = ih * iw
        a1 = (by2 - by1) * (bx2 - bx1)
        a2 = (y2_ref[...] - y1_ref[...]) * (x2_ref[...] - x1_ref[...])
        iou = inter / (a1 + a2 - inter + 1e-9)
        suppress = ((iou > IOU_THRESHOLD) & valid) | onehot
        sc_ref[...] = jnp.where(suppress, -1.0, sv)
        vf = valid.astype(jnp.float32)
        rowv = jnp.stack([by1, bx1, by2, bx2, blb, m]) * vf
        out_ref[0, pl.ds(i, 1), :] = rowv.reshape(1, 6)
        return 0

    lax.fori_loop(0, MAX_OUTPUT_SIZE, nms_iter, 0)


def _full_nms(preds, anc):
    batch = preds.shape[0]
    return pl.pallas_call(
        _full_nms_kernel,
        grid=(batch,),
        in_specs=[
            pl.BlockSpec((1, N, preds.shape[2]), lambda b: (b, 0, 0)),
            pl.BlockSpec((4, ROWS, LANES), lambda b: (0, 0, 0)),
        ],
        out_specs=pl.BlockSpec((1, MAX_OUTPUT_SIZE, 6), lambda b: (b, 0, 0)),
        out_shape=jax.ShapeDtypeStruct((batch, MAX_OUTPUT_SIZE, 6),
                                       jnp.float32),
        scratch_shapes=[pltpu.VMEM((ROWS, LANES), jnp.float32)
                        for _ in range(6)],
    )(preds, anc)


@jax.jit
def kernel(preds, anchors):
    anc = jnp.pad(anchors, ((0, NPAD - N), (0, 0))).T.reshape(4, ROWS, LANES)
    decoded = _tc_decode(preds, anc)
    flats = [a.reshape(-1) for a in decoded]
    sc_out, flags = _sc_select_nms(flats)
    sc_out = sc_out.reshape(4, MAX_OUTPUT_SIZE, 6)
    ok = jnp.all(flags.reshape(4, 8)[:, 0] == 0)
    return lax.cond(ok,
                    lambda ops: ops[0],
                    lambda ops: _full_nms(ops[1], ops[2]),
                    (sc_out, preds, anc))
